# baseline (speedup 1.0000x reference)
import jax
import jax.numpy as jnp
from jax import lax
from jax.experimental import pallas as pl
from jax.experimental.pallas import tpu as pltpu
from jax.experimental.pallas import tpu_sc as plsc

_NC, _NS, _L = 2, 16, 16
_NW = _NC * _NS
_NSEG = 50
_TPAD = 64
_MAGIC = 0x5F3759DF
_RPW = 24

_OFF = 8


def _build_tables(ws_v, pl_v, td_v, ta_v, tb_v, n_keys):
    iota = lax.iota(jnp.int32, _L)
    half = jnp.float32(0.5)
    one = jnp.full((_L,), 1, jnp.int32)
    zero = jnp.zeros((_L,), jnp.int32)
    for g in range(_TPAD // _L):
        kk = iota + (_L * g)
        x = kk.astype(jnp.float32) * half
        cnt = zero
        for j in range(n_keys):
            wj = plsc.load_gather(ws_v, [jnp.full((_L,), j + _OFF, jnp.int32)])
            cnt = cnt + jnp.where(wj <= x, one, zero)
        idx = jnp.clip(cnt, 1, n_keys - 1) + _OFF
        x0 = plsc.load_gather(ws_v, [idx - 1])
        x1 = plsc.load_gather(ws_v, [idx])
        y0 = plsc.load_gather(pl_v, [idx - 1])
        y1 = plsc.load_gather(pl_v, [idx])
        td_v[pl.ds(_OFF + _L * g, _L)] = y0 + (y1 - y0) * (x - x0) / (x1 - x0)
    for g in range(_TPAD // _L):
        kk = iota + (_L * g)
        t0 = plsc.load_gather(td_v, [jnp.minimum(kk, _NSEG) + _OFF])
        t1 = plsc.load_gather(td_v, [jnp.minimum(kk + 1, _NSEG) + _OFF])
        bb = t1 - t0
        ta_v[pl.ds(_L * g, _L)] = t0 - bb * kk.astype(jnp.float32)
        tb_v[pl.ds(_L * g, _L)] = bb


def _make_sc_call(h, w, n_keys):
    full_workers = h // _RPW
    rem = h - full_workers * _RPW
    csteps = w // _L
    assert w % _L == 0 and _RPW % 8 == 0 and full_workers + 1 <= _NW

    def _interp_rows(u_v, v_v, o_v, ta_v, tb_v, nrows):
        def row(r, carry):
            @plsc.parallel_loop(0, csteps, 1, unroll=5)
            def col(ci):
                sl = pl.ds(ci * _L, _L)
                u = u_v[r, sl]
                v = v_v[r, sl]
                s = jnp.maximum(u * u + v * v, jnp.float32(1e-30))
                y = lax.bitcast_convert_type(
                    jnp.int32(_MAGIC)
                    - lax.shift_right_logical(
                        lax.bitcast_convert_type(s, jnp.int32), 1),
                    jnp.float32)
                hh = jnp.float32(0.5) * s
                y = y * (jnp.float32(1.5) - hh * y * y)
                y = y * (jnp.float32(1.5) - hh * y * y)
                t = jnp.minimum((s * y) * jnp.float32(2.0), jnp.float32(_NSEG))
                k = jnp.minimum(t.astype(jnp.int32), _NSEG - 1)
                a = plsc.load_gather(ta_v, [k])
                b = plsc.load_gather(tb_v, [k])
                o_v[r, sl] = a + b * t
            return carry
        lax.fori_loop(0, nrows, row, 0)

    def body(uv_ref, ws_ref, pl_ref, out_ref,
             u_v, v_v, o_v, ws_v, pl_v, td_v, ta_v, tb_v, sem_u, sem_v):
        wid = lax.axis_index("c") * _NS + lax.axis_index("s")
        r0 = wid * _RPW

        @pl.when(wid < full_workers)
        def _():
            cp_u = pltpu.async_copy(
                uv_ref.at[0, pl.ds(r0, _RPW), :], u_v, sem_u)
            cp_v = pltpu.async_copy(
                uv_ref.at[1, pl.ds(r0, _RPW), :], v_v, sem_v)
            pltpu.sync_copy(ws_ref, ws_v.at[pl.ds(_OFF, n_keys)])
            pltpu.sync_copy(pl_ref, pl_v.at[pl.ds(_OFF, n_keys)])
            _build_tables(ws_v, pl_v, td_v, ta_v, tb_v, n_keys)
            cp_u.wait()
            cp_v.wait()
            _interp_rows(u_v, v_v, o_v, ta_v, tb_v, _RPW)
            pltpu.sync_copy(o_v, out_ref.at[0, pl.ds(r0, _RPW), :])

        if rem:
            @pl.when(wid == full_workers)
            def _():
                tr0 = full_workers * _RPW
                cp_u = pltpu.async_copy(
                    uv_ref.at[0, pl.ds(tr0, rem), :],
                    u_v.at[pl.ds(0, rem), :], sem_u)
                cp_v = pltpu.async_copy(
                    uv_ref.at[1, pl.ds(tr0, rem), :],
                    v_v.at[pl.ds(0, rem), :], sem_v)
                pltpu.sync_copy(ws_ref, ws_v.at[pl.ds(_OFF, n_keys)])
                pltpu.sync_copy(pl_ref, pl_v.at[pl.ds(_OFF, n_keys)])
                _build_tables(ws_v, pl_v, td_v, ta_v, tb_v, n_keys)
                cp_u.wait()
                cp_v.wait()
                _interp_rows(u_v, v_v, o_v, ta_v, tb_v, rem)
                pltpu.sync_copy(o_v.at[pl.ds(0, rem), :],
                                out_ref.at[0, pl.ds(tr0, rem), :])

    mesh = plsc.VectorSubcoreMesh(
        core_axis_name="c", subcore_axis_name="s",
        num_cores=_NC, num_subcores=_NS)
    return pl.kernel(
        body,
        out_type=jax.ShapeDtypeStruct((1, h, w), jnp.float32),
        mesh=mesh,
        scratch_types=[
            pltpu.VMEM((_RPW, w), jnp.float32),
            pltpu.VMEM((_RPW, w), jnp.float32),
            pltpu.VMEM((_RPW, w), jnp.float32),
            pltpu.VMEM((_OFF + 2 * _L,), jnp.float32),
            pltpu.VMEM((_OFF + 2 * _L,), jnp.float32),
            pltpu.VMEM((_OFF + _TPAD + _L,), jnp.float32),
            pltpu.VMEM((_TPAD,), jnp.float32),
            pltpu.VMEM((_TPAD,), jnp.float32),
            pltpu.SemaphoreType.DMA,
            pltpu.SemaphoreType.DMA,
        ],
        compiler_params=pltpu.CompilerParams(
            needs_layout_passes=False, use_tc_tiling_on_sc=True),
    )


def kernel(pangu_output_upper, pangu_output_surface, wind_speeds, power_levels):
    b, c, z, h, w = pangu_output_upper.shape
    uv = pangu_output_upper[0, 3:5, 0]
    n_keys = wind_speeds.shape[0]
    return _make_sc_call(h, w, n_keys)(uv, wind_speeds, power_levels)


# speedup vs baseline: 1.8613x; 1.8613x over previous
import jax
import jax.numpy as jnp
from jax import lax
from jax.experimental import pallas as pl
from jax.experimental.pallas import tpu as pltpu
from jax.experimental.pallas import tpu_sc as plsc

_NC, _NS, _L = 2, 16, 16
_NW = _NC * _NS
_NSEG = 50
_TPAD = 64
_MAGIC = 0x5F3759DF
_RPW = 24

_OFF = 8


def _build_tables(ws_v, pl_v, td_v, ta_v, tb_v, n_keys):
    iota = lax.iota(jnp.int32, _L)
    half = jnp.float32(0.5)
    one = jnp.full((_L,), 1, jnp.int32)
    zero = jnp.zeros((_L,), jnp.int32)
    for g in range(_TPAD // _L):
        kk = iota + (_L * g)
        x = kk.astype(jnp.float32) * half
        cnt = zero
        for j in range(n_keys):
            wj = plsc.load_gather(ws_v, [jnp.full((_L,), j + _OFF, jnp.int32)])
            cnt = cnt + jnp.where(wj <= x, one, zero)
        idx = jnp.clip(cnt, 1, n_keys - 1) + _OFF
        x0 = plsc.load_gather(ws_v, [idx - 1])
        x1 = plsc.load_gather(ws_v, [idx])
        y0 = plsc.load_gather(pl_v, [idx - 1])
        y1 = plsc.load_gather(pl_v, [idx])
        td_v[pl.ds(_OFF + _L * g, _L)] = y0 + (y1 - y0) * (x - x0) / (x1 - x0)
    for g in range(_TPAD // _L):
        kk = iota + (_L * g)
        t0 = plsc.load_gather(td_v, [jnp.minimum(kk, _NSEG) + _OFF])
        t1 = plsc.load_gather(td_v, [jnp.minimum(kk + 1, _NSEG) + _OFF])
        bb = t1 - t0
        ta_v[pl.ds(_L * g, _L)] = t0 - bb * kk.astype(jnp.float32)
        tb_v[pl.ds(_L * g, _L)] = bb


def _make_sc_call(h, w, n_keys):
    full_workers = h // _RPW
    rem = h - full_workers * _RPW
    csteps = w // _L
    assert w % _L == 0 and _RPW % 8 == 0 and full_workers + 1 <= _NW

    def _interp_rows(u_v, v_v, o_v, ta_v, tb_v, nrows):
        def row(r, carry):
            @plsc.parallel_loop(0, csteps, 1, unroll=5)
            def col(ci):
                sl = pl.ds(ci * _L, _L)
                u = u_v[r, sl]
                v = v_v[r, sl]
                s = jnp.maximum(u * u + v * v, jnp.float32(1e-30))
                y = lax.bitcast_convert_type(
                    jnp.int32(_MAGIC)
                    - lax.shift_right_logical(
                        lax.bitcast_convert_type(s, jnp.int32), 1),
                    jnp.float32)
                hh = jnp.float32(0.5) * s
                y = y * (jnp.float32(1.5) - hh * y * y)
                y = y * (jnp.float32(1.5) - hh * y * y)
                t = jnp.minimum((s * y) * jnp.float32(2.0), jnp.float32(_NSEG))
                k = jnp.minimum(t.astype(jnp.int32), _NSEG - 1)
                a = plsc.load_gather(ta_v, [k])
                b = plsc.load_gather(tb_v, [k])
                o_v[r, sl] = a + b * t
            return carry
        lax.fori_loop(0, nrows, row, 0)

    def body(u_ref, v_ref, ws_ref, pl_ref, out_ref,
             u_v, v_v, o_v, ws_v, pl_v, td_v, ta_v, tb_v, sem_u, sem_v):
        wid = lax.axis_index("c") * _NS + lax.axis_index("s")
        r0 = wid * _RPW

        @pl.when(wid < full_workers)
        def _():
            cp_u = pltpu.async_copy(u_ref.at[pl.ds(r0, _RPW), :], u_v, sem_u)
            cp_v = pltpu.async_copy(v_ref.at[pl.ds(r0, _RPW), :], v_v, sem_v)
            pltpu.sync_copy(ws_ref, ws_v.at[pl.ds(_OFF, n_keys)])
            pltpu.sync_copy(pl_ref, pl_v.at[pl.ds(_OFF, n_keys)])
            _build_tables(ws_v, pl_v, td_v, ta_v, tb_v, n_keys)
            cp_u.wait()
            cp_v.wait()
            _interp_rows(u_v, v_v, o_v, ta_v, tb_v, _RPW)
            pltpu.sync_copy(o_v, out_ref.at[0, pl.ds(r0, _RPW), :])

        if rem:
            @pl.when(wid == full_workers)
            def _():
                tr0 = full_workers * _RPW
                cp_u = pltpu.async_copy(
                    u_ref.at[pl.ds(tr0, rem), :],
                    u_v.at[pl.ds(0, rem), :], sem_u)
                cp_v = pltpu.async_copy(
                    v_ref.at[pl.ds(tr0, rem), :],
                    v_v.at[pl.ds(0, rem), :], sem_v)
                pltpu.sync_copy(ws_ref, ws_v.at[pl.ds(_OFF, n_keys)])
                pltpu.sync_copy(pl_ref, pl_v.at[pl.ds(_OFF, n_keys)])
                _build_tables(ws_v, pl_v, td_v, ta_v, tb_v, n_keys)
                cp_u.wait()
                cp_v.wait()
                _interp_rows(u_v, v_v, o_v, ta_v, tb_v, rem)
                pltpu.sync_copy(o_v.at[pl.ds(0, rem), :],
                                out_ref.at[0, pl.ds(tr0, rem), :])

    mesh = plsc.VectorSubcoreMesh(
        core_axis_name="c", subcore_axis_name="s",
        num_cores=_NC, num_subcores=_NS)
    return pl.kernel(
        body,
        out_type=jax.ShapeDtypeStruct((1, h, w), jnp.float32),
        mesh=mesh,
        scratch_types=[
            pltpu.VMEM((_RPW, w), jnp.float32),
            pltpu.VMEM((_RPW, w), jnp.float32),
            pltpu.VMEM((_RPW, w), jnp.float32),
            pltpu.VMEM((_OFF + 2 * _L,), jnp.float32),
            pltpu.VMEM((_OFF + 2 * _L,), jnp.float32),
            pltpu.VMEM((_OFF + _TPAD + _L,), jnp.float32),
            pltpu.VMEM((_TPAD,), jnp.float32),
            pltpu.VMEM((_TPAD,), jnp.float32),
            pltpu.SemaphoreType.DMA,
            pltpu.SemaphoreType.DMA,
        ],
        compiler_params=pltpu.CompilerParams(
            needs_layout_passes=False, use_tc_tiling_on_sc=True),
    )


def kernel(pangu_output_upper, pangu_output_surface, wind_speeds, power_levels):
    b, c, z, h, w = pangu_output_upper.shape
    u2 = pangu_output_upper[0, 3, 0]
    v2 = pangu_output_upper[0, 4, 0]
    n_keys = wind_speeds.shape[0]
    return _make_sc_call(h, w, n_keys)(u2, v2, wind_speeds, power_levels)


# speedup vs baseline: 1.8642x; 1.0015x over previous
import jax
import jax.numpy as jnp
from jax import lax
from jax.experimental import pallas as pl
from jax.experimental.pallas import tpu as pltpu
from jax.experimental.pallas import tpu_sc as plsc

_NC, _NS, _L = 2, 16, 16
_NW = _NC * _NS
_NSEG = 50
_TPAD = 64
_MAGIC = 0x5F3759DF
_RPW = 24

_OFF = 8


def _build_tables(ws_v, pl_v, td_v, ta_v, tb_v, n_keys):
    iota = lax.iota(jnp.int32, _L)
    half = jnp.float32(0.5)
    one = jnp.full((_L,), 1, jnp.int32)
    zero = jnp.zeros((_L,), jnp.int32)
    for g in range(_TPAD // _L):
        kk = iota + (_L * g)
        x = kk.astype(jnp.float32) * half
        cnt = zero
        for j in range(n_keys):
            wj = plsc.load_gather(ws_v, [jnp.full((_L,), j + _OFF, jnp.int32)])
            cnt = cnt + jnp.where(wj <= x, one, zero)
        idx = jnp.clip(cnt, 1, n_keys - 1) + _OFF
        x0 = plsc.load_gather(ws_v, [idx - 1])
        x1 = plsc.load_gather(ws_v, [idx])
        y0 = plsc.load_gather(pl_v, [idx - 1])
        y1 = plsc.load_gather(pl_v, [idx])
        td_v[pl.ds(_OFF + _L * g, _L)] = y0 + (y1 - y0) * (x - x0) / (x1 - x0)
    for g in range(_TPAD // _L):
        kk = iota + (_L * g)
        t0 = plsc.load_gather(td_v, [jnp.minimum(kk, _NSEG) + _OFF])
        t1 = plsc.load_gather(td_v, [jnp.minimum(kk + 1, _NSEG) + _OFF])
        bb = t1 - t0
        ta_v[pl.ds(_L * g, _L)] = t0 - bb * kk.astype(jnp.float32)
        tb_v[pl.ds(_L * g, _L)] = bb


def _make_sc_call(h, w, n_keys):
    full_workers = h // _RPW
    rem = h - full_workers * _RPW
    csteps = w // _L
    assert w % _L == 0 and _RPW % 8 == 0 and full_workers + 1 <= _NW

    def _interp_rows(u_v, v_v, o_v, ta_v, tb_v, nrows):
        def row(r, carry):
            @plsc.parallel_loop(0, csteps, 1, unroll=5)
            def col(ci):
                sl = pl.ds(ci * _L, _L)
                u = u_v[r, sl]
                v = v_v[r, sl]
                s = jnp.maximum(u * u + v * v, jnp.float32(1e-30))
                y = lax.bitcast_convert_type(
                    jnp.int32(_MAGIC)
                    - lax.shift_right_logical(
                        lax.bitcast_convert_type(s, jnp.int32), 1),
                    jnp.float32)
                hh = jnp.float32(0.5) * s
                y = y * (jnp.float32(1.5) - hh * y * y)
                y = y * (jnp.float32(1.5) - hh * y * y)
                t = jnp.minimum((s * y) * jnp.float32(2.0), jnp.float32(_NSEG))
                k = jnp.minimum(t.astype(jnp.int32), _NSEG - 1)
                a = plsc.load_gather(ta_v, [k])
                b = plsc.load_gather(tb_v, [k])
                o_v[r, sl] = a + b * t
            return carry
        lax.fori_loop(0, nrows, row, 0)

    def body(u_ref, v_ref, ws_ref, pl_ref, out_ref,
             u_v, v_v, o_v, ws_v, pl_v, td_v, ta_v, tb_v, sem_u, sem_v):
        wid = lax.axis_index("c") * _NS + lax.axis_index("s")
        r0 = wid * _RPW

        @pl.when(wid < full_workers)
        def _():
            cp_u = pltpu.async_copy(u_ref.at[pl.ds(r0, _RPW), :], u_v, sem_u)
            cp_v = pltpu.async_copy(v_ref.at[pl.ds(r0, _RPW), :], v_v, sem_v)
            pltpu.sync_copy(ws_ref, ws_v.at[pl.ds(_OFF, n_keys)])
            pltpu.sync_copy(pl_ref, pl_v.at[pl.ds(_OFF, n_keys)])
            _build_tables(ws_v, pl_v, td_v, ta_v, tb_v, n_keys)
            cp_u.wait()
            cp_v.wait()
            _interp_rows(u_v, v_v, o_v, ta_v, tb_v, _RPW)
            pltpu.sync_copy(o_v, out_ref.at[pl.ds(r0, _RPW), :])

        if rem:
            @pl.when(wid == full_workers)
            def _():
                tr0 = full_workers * _RPW
                cp_u = pltpu.async_copy(
                    u_ref.at[pl.ds(tr0, rem), :],
                    u_v.at[pl.ds(0, rem), :], sem_u)
                cp_v = pltpu.async_copy(
                    v_ref.at[pl.ds(tr0, rem), :],
                    v_v.at[pl.ds(0, rem), :], sem_v)
                pltpu.sync_copy(ws_ref, ws_v.at[pl.ds(_OFF, n_keys)])
                pltpu.sync_copy(pl_ref, pl_v.at[pl.ds(_OFF, n_keys)])
                _build_tables(ws_v, pl_v, td_v, ta_v, tb_v, n_keys)
                cp_u.wait()
                cp_v.wait()
                _interp_rows(u_v, v_v, o_v, ta_v, tb_v, rem)
                pltpu.sync_copy(o_v.at[pl.ds(0, rem), :],
                                out_ref.at[pl.ds(tr0, rem), :])

    mesh = plsc.VectorSubcoreMesh(
        core_axis_name="c", subcore_axis_name="s",
        num_cores=_NC, num_subcores=_NS)
    return pl.kernel(
        body,
        out_type=jax.ShapeDtypeStruct((h, w), jnp.float32),
        mesh=mesh,
        scratch_types=[
            pltpu.VMEM((_RPW, w), jnp.float32),
            pltpu.VMEM((_RPW, w), jnp.float32),
            pltpu.VMEM((_RPW, w), jnp.float32),
            pltpu.VMEM((_OFF + 2 * _L,), jnp.float32),
            pltpu.VMEM((_OFF + 2 * _L,), jnp.float32),
            pltpu.VMEM((_OFF + _TPAD + _L,), jnp.float32),
            pltpu.VMEM((_TPAD,), jnp.float32),
            pltpu.VMEM((_TPAD,), jnp.float32),
            pltpu.SemaphoreType.DMA,
            pltpu.SemaphoreType.DMA,
        ],
        compiler_params=pltpu.CompilerParams(
            needs_layout_passes=False, use_tc_tiling_on_sc=True),
    )


def kernel(pangu_output_upper, pangu_output_surface, wind_speeds, power_levels):
    b, c, z, h, w = pangu_output_upper.shape
    u2 = pangu_output_upper[0, 3, 0]
    v2 = pangu_output_upper[0, 4, 0]
    n_keys = wind_speeds.shape[0]
    out = _make_sc_call(h, w, n_keys)(u2, v2, wind_speeds, power_levels)
    return out.reshape(1, h, w)
